# baseline (speedup 1.0000x reference)
import jax
import jax.numpy as jnp
from jax import lax
from jax.experimental import pallas as pl
from jax.experimental.pallas import tpu as pltpu
from jax.experimental.pallas import tpu_sc as plsc

N = 10000
E = 320000
D_IN = 128
D_HID = 128
D_OUT = 40
D_OUT_PAD = 48

CH = 128
NCHUNK = E // CH
NC = 2
NS = 16
NW = NC * NS
CPW = NCHUNK // NW
TAIL = NCHUNK - CPW * NW

RT = 624
RT_LAST = N - RT * (NS - 1)


def _make_agg(D):
  mesh = plsc.VectorSubcoreMesh(core_axis_name="c", subcore_axis_name="s")

  def body(hw, ei, zeros, part,
           isv, idv, rows0, rows1, rows2, acc,
           ig0, ig1, ig2, ig3, ig4, ig5,
           gs0, gs1, gs2, ss0, ss1, ss2):
    cid = lax.axis_index("c")
    sid = lax.axis_index("s")
    wid = sid * NC + cid
    c0 = wid * CPW

    base = sid * RT

    @pl.when(sid < NS - 1)
    def _():
      pltpu.sync_copy(zeros.at[pl.ds(base, RT)], acc.at[pl.ds(base, RT)])

    @pl.when(sid == NS - 1)
    def _():
      pltpu.sync_copy(zeros.at[pl.ds(base, RT_LAST)], acc.at[pl.ds(base, RT_LAST)])

    plsc.subcore_barrier()

    rows = (rows0, rows1, rows2)
    gsem = (gs0, gs1, gs2)
    ssem = (ss0, ss1, ss2)
    isem = (ig0, ig1, ig2, ig3, ig4, ig5)

    def fire_i(c, q):
      pltpu.async_copy(ei.at[0, pl.ds(c * CH, CH)], isv.at[q], isem[q])
      pltpu.async_copy(ei.at[1, pl.ds(c * CH, CH)], idv.at[q], isem[q])

    def wait_i(q):
      pltpu.make_async_copy(ei.at[0, pl.ds(0, CH)], isv.at[q], isem[q]).wait()
      pltpu.make_async_copy(ei.at[1, pl.ds(0, CH)], idv.at[q], isem[q]).wait()

    def fire_g(q, b):
      pltpu.async_copy(hw.at[isv.at[q]], rows[b], gsem[b])

    def wait_g(b):
      pltpu.make_async_copy(hw.at[pl.ds(0, CH)], rows[b], gsem[b]).wait()

    def fire_s(q, b):
      pltpu.async_copy(rows[b], acc.at[idv.at[q]], ssem[b], add=True)

    def wait_s(b):
      pltpu.make_async_copy(rows[b], acc.at[pl.ds(0, CH)], ssem[b]).wait()

    for q in range(4):
      fire_i(c0 + q, q)
    wait_i(0)
    fire_g(0, 0)
    wait_i(1)
    fire_g(1, 1)

    def step(u, carry):
      for m in range(6):
        j = u * 6 + m
        b = m % 3

        wait_g(b)
        fire_s(m, b)

        @pl.when(j > 0)
        def _():
          wait_s((m + 2) % 3)

        @pl.when(j + 2 < CPW)
        def _():
          wait_i((m + 2) % 6)
          fire_g((m + 2) % 6, (m + 2) % 3)

        @pl.when(j + 4 < CPW)
        def _():
          fire_i(c0 + j + 4, (m + 4) % 6)

      return carry

    lax.fori_loop(0, CPW // 6, step, 0, unroll=False)
    wait_s((CPW - 1) % 3)

    @pl.when(wid < TAIL)
    def _():
      fire_i(NW * CPW + wid, 0)
      wait_i(0)
      fire_g(0, 0)
      wait_g(0)
      fire_s(0, 0)
      wait_s(0)

    plsc.subcore_barrier()

    @pl.when(sid < NS - 1)
    def _():
      pltpu.sync_copy(acc.at[pl.ds(base, RT)], part.at[cid, pl.ds(base, RT)])

    @pl.when(sid == NS - 1)
    def _():
      pltpu.sync_copy(acc.at[pl.ds(base, RT_LAST)],
                      part.at[cid, pl.ds(base, RT_LAST)])

  return pl.kernel(
      body,
      out_type=jax.ShapeDtypeStruct((NC, N, D), jnp.float32),
      mesh=mesh,
      compiler_params=pltpu.CompilerParams(use_tc_tiling_on_sc=False),
      scratch_types=(
          [pltpu.VMEM((6, CH), jnp.int32),
           pltpu.VMEM((6, CH), jnp.int32),
           pltpu.VMEM((CH, D), jnp.float32),
           pltpu.VMEM((CH, D), jnp.float32),
           pltpu.VMEM((CH, D), jnp.float32),
           pltpu.VMEM_SHARED((N, D), jnp.float32)]
          + [pltpu.SemaphoreType.DMA] * 12),
  )


def _make_agg_slab(D, K):
  NSLAB = CPW // K
  assert NSLAB % 3 == 0
  mesh = plsc.VectorSubcoreMesh(core_axis_name="c", subcore_axis_name="s")

  def body(hw, ei, zeros, part,
           isv, idv, rows0, rows1, rows2, acc,
           ig0, ig1, ig2, gs0, gs1, gs2, ss0, ss1, ss2):
    cid = lax.axis_index("c")
    sid = lax.axis_index("s")
    wid = sid * NC + cid
    c0 = wid * CPW

    base = sid * RT

    @pl.when(sid < NS - 1)
    def _():
      pltpu.sync_copy(zeros.at[pl.ds(base, RT)], acc.at[pl.ds(base, RT)])

    @pl.when(sid == NS - 1)
    def _():
      pltpu.sync_copy(zeros.at[pl.ds(base, RT_LAST)], acc.at[pl.ds(base, RT_LAST)])

    plsc.subcore_barrier()

    rows = (rows0, rows1, rows2)
    gsem = (gs0, gs1, gs2)
    ssem = (ss0, ss1, ss2)
    isem = (ig0, ig1, ig2)

    def fire_i(s, r):
      for k in range(K):
        c = c0 + s * K + k
        pltpu.async_copy(ei.at[0, pl.ds(c * CH, CH)], isv.at[r * K + k], isem[r])
        pltpu.async_copy(ei.at[1, pl.ds(c * CH, CH)], idv.at[r * K + k], isem[r])

    def wait_i(r):
      for k in range(K):
        pltpu.make_async_copy(ei.at[0, pl.ds(0, CH)], isv.at[r * K + k], isem[r]).wait()
        pltpu.make_async_copy(ei.at[1, pl.ds(0, CH)], idv.at[r * K + k], isem[r]).wait()

    def fire_g(r):
      for k in range(K):
        pltpu.async_copy(hw.at[isv.at[r * K + k]], rows[r].at[pl.ds(k * CH, CH)],
                         gsem[r])

    def wait_g(r):
      pltpu.make_async_copy(hw.at[pl.ds(0, K * CH)], rows[r], gsem[r]).wait()

    def fire_s(r):
      for k in range(K):
        pltpu.async_copy(rows[r].at[pl.ds(k * CH, CH)], acc.at[idv.at[r * K + k]],
                         ssem[r], add=True)

    def wait_s(r):
      pltpu.make_async_copy(rows[r], acc.at[pl.ds(0, K * CH)], ssem[r]).wait()

    fire_i(0, 0)
    fire_i(1, 1)
    wait_i(0)
    fire_g(0)

    def step(u, carry):
      for m in range(3):
        s = u * 3 + m

        wait_g(m)
        fire_s(m)

        @pl.when(s > 0)
        def _():
          wait_s((m + 2) % 3)

        @pl.when(s + 1 < NSLAB)
        def _():
          wait_i((m + 1) % 3)
          fire_g((m + 1) % 3)

        @pl.when(s + 2 < NSLAB)
        def _():
          fire_i(s + 2, (m + 2) % 3)

      return carry

    lax.fori_loop(0, NSLAB // 3, step, 0, unroll=False)
    wait_s((NSLAB - 1) % 3)

    @pl.when(wid < TAIL)
    def _():
      c = NW * CPW + wid
      pltpu.async_copy(ei.at[0, pl.ds(c * CH, CH)], isv.at[0], isem[0])
      pltpu.async_copy(ei.at[1, pl.ds(c * CH, CH)], idv.at[0], isem[0])
      pltpu.make_async_copy(ei.at[0, pl.ds(0, CH)], isv.at[0], isem[0]).wait()
      pltpu.make_async_copy(ei.at[1, pl.ds(0, CH)], idv.at[0], isem[0]).wait()
      pltpu.async_copy(hw.at[isv.at[0]], rows0.at[pl.ds(0, CH)], gs0)
      pltpu.make_async_copy(hw.at[pl.ds(0, CH)], rows0.at[pl.ds(0, CH)], gs0).wait()
      pltpu.async_copy(rows0.at[pl.ds(0, CH)], acc.at[idv.at[0]], ss0, add=True)
      pltpu.make_async_copy(rows0.at[pl.ds(0, CH)], acc.at[pl.ds(0, CH)], ss0).wait()

    plsc.subcore_barrier()

    @pl.when(sid < NS - 1)
    def _():
      pltpu.sync_copy(acc.at[pl.ds(base, RT)], part.at[cid, pl.ds(base, RT)])

    @pl.when(sid == NS - 1)
    def _():
      pltpu.sync_copy(acc.at[pl.ds(base, RT_LAST)],
                      part.at[cid, pl.ds(base, RT_LAST)])

  return pl.kernel(
      body,
      out_type=jax.ShapeDtypeStruct((NC, N, D), jnp.float32),
      mesh=mesh,
      compiler_params=pltpu.CompilerParams(use_tc_tiling_on_sc=False),
      scratch_types=(
          [pltpu.VMEM((3 * K, CH), jnp.int32),
           pltpu.VMEM((3 * K, CH), jnp.int32),
           pltpu.VMEM((K * CH, D), jnp.float32),
           pltpu.VMEM((K * CH, D), jnp.float32),
           pltpu.VMEM((K * CH, D), jnp.float32),
           pltpu.VMEM_SHARED((N, D), jnp.float32)]
          + [pltpu.SemaphoreType.DMA] * 9),
  )


_agg128 = _make_agg(D_HID)
_agg48 = _make_agg_slab(D_OUT_PAD, 2)

_BM = 1000


def _mm_body(x_ref, w_ref, o_ref):
  o_ref[...] = jnp.dot(x_ref[...], w_ref[...], preferred_element_type=jnp.float32)


def _mm(x, W):
  M, K = x.shape
  Dw = W.shape[1]
  return pl.pallas_call(
      _mm_body,
      grid=(M // _BM,),
      in_specs=[pl.BlockSpec((_BM, K), lambda i: (i, 0)),
                pl.BlockSpec((K, Dw), lambda i: (0, 0))],
      out_specs=pl.BlockSpec((_BM, Dw), lambda i: (i, 0)),
      out_shape=jax.ShapeDtypeStruct((M, Dw), jnp.float32),
  )(x, W)


def _cmb_mm_body(p_ref, b_ref, w_ref, o_ref):
  g = jnp.maximum(p_ref[0] + p_ref[1] + b_ref[...], 0.0)
  o_ref[...] = jnp.dot(g, w_ref[...], preferred_element_type=jnp.float32)


def _cmb_mm(p, b, W):
  K = p.shape[2]
  Dw = W.shape[1]
  return pl.pallas_call(
      _cmb_mm_body,
      grid=(N // _BM,),
      in_specs=[pl.BlockSpec((NC, _BM, K), lambda i: (0, i, 0)),
                pl.BlockSpec((1, K), lambda i: (0, 0)),
                pl.BlockSpec((K, Dw), lambda i: (0, 0))],
      out_specs=pl.BlockSpec((_BM, Dw), lambda i: (i, 0)),
      out_shape=jax.ShapeDtypeStruct((N, Dw), jnp.float32),
  )(p, b, W)


def _fin_body(q_ref, b_ref, o_ref):
  s = q_ref[0] + q_ref[1]
  o_ref[...] = s[:, :D_OUT] + b_ref[...]


def _fin(q, b):
  return pl.pallas_call(
      _fin_body,
      grid=(N // _BM,),
      in_specs=[pl.BlockSpec((NC, _BM, D_OUT_PAD), lambda i: (0, i, 0)),
                pl.BlockSpec((1, D_OUT), lambda i: (0, 0))],
      out_specs=pl.BlockSpec((_BM, D_OUT), lambda i: (i, 0)),
      out_shape=jax.ShapeDtypeStruct((N, D_OUT), jnp.float32),
  )(q, b)


def kernel(x, edge_index, label_p, cm, W1, b1, W2, b2, W3, b3):
  z128 = jnp.zeros((N, D_HID), jnp.float32)
  z48 = jnp.zeros((N, D_OUT_PAD), jnp.float32)
  W3p = jnp.pad(W3, ((0, 0), (0, D_OUT_PAD - D_OUT)))

  hw1 = _mm(x, W1)
  p1 = _agg128(hw1, edge_index, z128)
  hw2 = _cmb_mm(p1, b1.reshape(1, -1), W2)
  p2 = _agg128(hw2, edge_index, z128)
  hw3 = _cmb_mm(p2, b2.reshape(1, -1), W3p)
  q = _agg48(hw3, edge_index, z48)
  return _fin(q, b3.reshape(1, -1))


# speedup vs baseline: 1.0241x; 1.0241x over previous
import jax
import jax.numpy as jnp
from jax import lax
from jax.experimental import pallas as pl
from jax.experimental.pallas import tpu as pltpu
from jax.experimental.pallas import tpu_sc as plsc

N = 10000
E = 320000
D_IN = 128
D_HID = 128
D_OUT = 40
D_OUT_PAD = 48

CH = 128
NCHUNK = E // CH
NC = 2
NS = 16
NW = NC * NS
CPW = NCHUNK // NW
TAIL = NCHUNK - CPW * NW

RT = 624
RT_LAST = N - RT * (NS - 1)


def _make_agg(D):
  mesh = plsc.VectorSubcoreMesh(core_axis_name="c", subcore_axis_name="s")

  def body(hw, ei, zeros, part,
           isv, idv, rows0, rows1, rows2, acc,
           ig0, ig1, ig2, ig3, ig4, ig5,
           gs0, gs1, gs2, ss0, ss1, ss2):
    cid = lax.axis_index("c")
    sid = lax.axis_index("s")
    wid = sid * NC + cid
    c0 = wid * CPW

    base = sid * RT

    @pl.when(sid < NS - 1)
    def _():
      pltpu.sync_copy(zeros.at[pl.ds(base, RT)], acc.at[pl.ds(base, RT)])

    @pl.when(sid == NS - 1)
    def _():
      pltpu.sync_copy(zeros.at[pl.ds(base, RT_LAST)], acc.at[pl.ds(base, RT_LAST)])

    plsc.subcore_barrier()

    rows = (rows0, rows1, rows2)
    gsem = (gs0, gs1, gs2)
    ssem = (ss0, ss1, ss2)
    isem = (ig0, ig1, ig2, ig3, ig4, ig5)

    def fire_i(c, q):
      pltpu.async_copy(ei.at[0, pl.ds(c * CH, CH)], isv.at[q], isem[q])
      pltpu.async_copy(ei.at[1, pl.ds(c * CH, CH)], idv.at[q], isem[q])

    def wait_i(q):
      pltpu.make_async_copy(ei.at[0, pl.ds(0, CH)], isv.at[q], isem[q]).wait()
      pltpu.make_async_copy(ei.at[1, pl.ds(0, CH)], idv.at[q], isem[q]).wait()

    def fire_g(q, b):
      pltpu.async_copy(hw.at[isv.at[q]], rows[b], gsem[b])

    def wait_g(b):
      pltpu.make_async_copy(hw.at[pl.ds(0, CH)], rows[b], gsem[b]).wait()

    def fire_s(q, b):
      pltpu.async_copy(rows[b], acc.at[idv.at[q]], ssem[b], add=True)

    def wait_s(b):
      pltpu.make_async_copy(rows[b], acc.at[pl.ds(0, CH)], ssem[b]).wait()

    for q in range(4):
      fire_i(c0 + q, q)
    wait_i(0)
    fire_g(0, 0)
    wait_i(1)
    fire_g(1, 1)

    def step(u, carry):
      for m in range(6):
        j = u * 6 + m
        b = m % 3

        wait_g(b)
        fire_s(m, b)

        @pl.when(j + 4 < CPW)
        def _():
          fire_i(c0 + j + 4, (m + 4) % 6)

        @pl.when(j > 0)
        def _():
          wait_s((m + 2) % 3)

        @pl.when(j + 2 < CPW)
        def _():
          wait_i((m + 2) % 6)
          fire_g((m + 2) % 6, (m + 2) % 3)

      return carry

    lax.fori_loop(0, CPW // 6, step, 0, unroll=False)
    wait_s((CPW - 1) % 3)

    @pl.when(wid < TAIL)
    def _():
      fire_i(NW * CPW + wid, 0)
      wait_i(0)
      fire_g(0, 0)
      wait_g(0)
      fire_s(0, 0)
      wait_s(0)

    plsc.subcore_barrier()

    @pl.when(sid < NS - 1)
    def _():
      pltpu.sync_copy(acc.at[pl.ds(base, RT)], part.at[cid, pl.ds(base, RT)])

    @pl.when(sid == NS - 1)
    def _():
      pltpu.sync_copy(acc.at[pl.ds(base, RT_LAST)],
                      part.at[cid, pl.ds(base, RT_LAST)])

  return pl.kernel(
      body,
      out_type=jax.ShapeDtypeStruct((NC, N, D), jnp.float32),
      mesh=mesh,
      compiler_params=pltpu.CompilerParams(use_tc_tiling_on_sc=False),
      scratch_types=(
          [pltpu.VMEM((6, CH), jnp.int32),
           pltpu.VMEM((6, CH), jnp.int32),
           pltpu.VMEM((CH, D), jnp.float32),
           pltpu.VMEM((CH, D), jnp.float32),
           pltpu.VMEM((CH, D), jnp.float32),
           pltpu.VMEM_SHARED((N, D), jnp.float32)]
          + [pltpu.SemaphoreType.DMA] * 12),
  )


_agg128 = _make_agg(D_HID)
_agg48 = _make_agg(D_OUT_PAD)

_BM = 1000


def _mm_body(x_ref, w_ref, o_ref):
  o_ref[...] = jnp.dot(x_ref[...], w_ref[...], preferred_element_type=jnp.float32)


def _mm(x, W):
  M, K = x.shape
  Dw = W.shape[1]
  return pl.pallas_call(
      _mm_body,
      grid=(M // _BM,),
      in_specs=[pl.BlockSpec((_BM, K), lambda i: (i, 0)),
                pl.BlockSpec((K, Dw), lambda i: (0, 0))],
      out_specs=pl.BlockSpec((_BM, Dw), lambda i: (i, 0)),
      out_shape=jax.ShapeDtypeStruct((M, Dw), jnp.float32),
  )(x, W)


def _cmb_mm_body(p_ref, b_ref, w_ref, o_ref):
  g = jnp.maximum(p_ref[0] + p_ref[1] + b_ref[...], 0.0)
  o_ref[...] = jnp.dot(g, w_ref[...], preferred_element_type=jnp.float32)


def _cmb_mm(p, b, W):
  K = p.shape[2]
  Dw = W.shape[1]
  return pl.pallas_call(
      _cmb_mm_body,
      grid=(N // _BM,),
      in_specs=[pl.BlockSpec((NC, _BM, K), lambda i: (0, i, 0)),
                pl.BlockSpec((1, K), lambda i: (0, 0)),
                pl.BlockSpec((K, Dw), lambda i: (0, 0))],
      out_specs=pl.BlockSpec((_BM, Dw), lambda i: (i, 0)),
      out_shape=jax.ShapeDtypeStruct((N, Dw), jnp.float32),
  )(p, b, W)


def _fin_body(q_ref, b_ref, o_ref):
  s = q_ref[0] + q_ref[1]
  o_ref[...] = s[:, :D_OUT] + b_ref[...]


def _fin(q, b):
  return pl.pallas_call(
      _fin_body,
      grid=(N // _BM,),
      in_specs=[pl.BlockSpec((NC, _BM, D_OUT_PAD), lambda i: (0, i, 0)),
                pl.BlockSpec((1, D_OUT), lambda i: (0, 0))],
      out_specs=pl.BlockSpec((_BM, D_OUT), lambda i: (i, 0)),
      out_shape=jax.ShapeDtypeStruct((N, D_OUT), jnp.float32),
  )(q, b)


def kernel(x, edge_index, label_p, cm, W1, b1, W2, b2, W3, b3):
  z128 = jnp.zeros((N, D_HID), jnp.float32)
  z48 = jnp.zeros((N, D_OUT_PAD), jnp.float32)
  W3p = jnp.pad(W3, ((0, 0), (0, D_OUT_PAD - D_OUT)))

  hw1 = _mm(x, W1)
  p1 = _agg128(hw1, edge_index, z128)
  hw2 = _cmb_mm(p1, b1.reshape(1, -1), W2)
  p2 = _agg128(hw2, edge_index, z128)
  hw3 = _cmb_mm(p2, b2.reshape(1, -1), W3p)
  q = _agg48(hw3, edge_index, z48)
  return _fin(q, b3.reshape(1, -1))


# speedup vs baseline: 1.0359x; 1.0116x over previous
import jax
import jax.numpy as jnp
from jax import lax
from jax.experimental import pallas as pl
from jax.experimental.pallas import tpu as pltpu
from jax.experimental.pallas import tpu_sc as plsc

N = 10000
E = 320000
D_IN = 128
D_HID = 128
D_OUT = 40
D_OUT_PAD = 48

CH = 128
NCHUNK = E // CH
NC = 2
NS = 16
NW = NC * NS
CPW = NCHUNK // NW
TAIL = NCHUNK - CPW * NW

RT = 624
RT_LAST = N - RT * (NS - 1)


def _make_agg(D):
  mesh = plsc.VectorSubcoreMesh(core_axis_name="c", subcore_axis_name="s")

  def body(hw, ei, zeros, part,
           isv, idv, rows0, rows1, rows2, acc,
           ig0, ig1, ig2, ig3, ig4, ig5,
           gs0, gs1, gs2, ss0, ss1, ss2):
    cid = lax.axis_index("c")
    sid = lax.axis_index("s")
    wid = sid * NC + cid
    c0 = wid * CPW

    base = sid * RT
    rows = (rows0, rows1, rows2)
    gsem = (gs0, gs1, gs2)
    ssem = (ss0, ss1, ss2)
    isem = (ig0, ig1, ig2, ig3, ig4, ig5)

    def fire_i(c, q):
      pltpu.async_copy(ei.at[0, pl.ds(c * CH, CH)], isv.at[q], isem[q])
      pltpu.async_copy(ei.at[1, pl.ds(c * CH, CH)], idv.at[q], isem[q])

    def wait_i(q):
      pltpu.make_async_copy(ei.at[0, pl.ds(0, CH)], isv.at[q], isem[q]).wait()
      pltpu.make_async_copy(ei.at[1, pl.ds(0, CH)], idv.at[q], isem[q]).wait()

    def fire_g(q, b):
      pltpu.async_copy(hw.at[isv.at[q]], rows[b], gsem[b])

    def wait_g(b):
      pltpu.make_async_copy(hw.at[pl.ds(0, CH)], rows[b], gsem[b]).wait()

    def fire_s(q, b):
      pltpu.async_copy(rows[b], acc.at[idv.at[q]], ssem[b], add=True)

    def wait_s(b):
      pltpu.make_async_copy(rows[b], acc.at[pl.ds(0, CH)], ssem[b]).wait()

    for q in range(4):
      fire_i(c0 + q, q)
    wait_i(0)
    fire_g(0, 0)
    wait_i(1)
    fire_g(1, 1)

    @pl.when(sid < NS - 1)
    def _():
      pltpu.sync_copy(zeros.at[pl.ds(base, RT)], acc.at[pl.ds(base, RT)])

    @pl.when(sid == NS - 1)
    def _():
      pltpu.sync_copy(zeros.at[pl.ds(base, RT_LAST)], acc.at[pl.ds(base, RT_LAST)])

    plsc.subcore_barrier()

    def step(u, carry):
      for m in range(6):
        j = u * 6 + m
        b = m % 3

        wait_g(b)
        fire_s(m, b)

        @pl.when(j + 4 < CPW)
        def _():
          fire_i(c0 + j + 4, (m + 4) % 6)

        @pl.when(j > 0)
        def _():
          wait_s((m + 2) % 3)

        @pl.when(j + 2 < CPW)
        def _():
          wait_i((m + 2) % 6)
          fire_g((m + 2) % 6, (m + 2) % 3)

      return carry

    lax.fori_loop(0, CPW // 6, step, 0, unroll=False)
    wait_s((CPW - 1) % 3)

    @pl.when(wid < TAIL)
    def _():
      fire_i(NW * CPW + wid, 0)
      wait_i(0)
      fire_g(0, 0)
      wait_g(0)
      fire_s(0, 0)
      wait_s(0)

    plsc.subcore_barrier()

    @pl.when(sid < NS - 1)
    def _():
      pltpu.sync_copy(acc.at[pl.ds(base, RT)], part.at[cid, pl.ds(base, RT)])

    @pl.when(sid == NS - 1)
    def _():
      pltpu.sync_copy(acc.at[pl.ds(base, RT_LAST)],
                      part.at[cid, pl.ds(base, RT_LAST)])

  return pl.kernel(
      body,
      out_type=jax.ShapeDtypeStruct((NC, N, D), jnp.float32),
      mesh=mesh,
      compiler_params=pltpu.CompilerParams(use_tc_tiling_on_sc=False),
      scratch_types=(
          [pltpu.VMEM((6, CH), jnp.int32),
           pltpu.VMEM((6, CH), jnp.int32),
           pltpu.VMEM((CH, D), jnp.float32),
           pltpu.VMEM((CH, D), jnp.float32),
           pltpu.VMEM((CH, D), jnp.float32),
           pltpu.VMEM_SHARED((N, D), jnp.float32)]
          + [pltpu.SemaphoreType.DMA] * 12),
  )


_agg128 = _make_agg(D_HID)
_agg48 = _make_agg(D_OUT_PAD)

_BM = 1000


def _mm_body(x_ref, w_ref, o_ref):
  o_ref[...] = jnp.dot(x_ref[...], w_ref[...], preferred_element_type=jnp.float32)


def _mm(x, W):
  M, K = x.shape
  Dw = W.shape[1]
  return pl.pallas_call(
      _mm_body,
      grid=(M // _BM,),
      in_specs=[pl.BlockSpec((_BM, K), lambda i: (i, 0)),
                pl.BlockSpec((K, Dw), lambda i: (0, 0))],
      out_specs=pl.BlockSpec((_BM, Dw), lambda i: (i, 0)),
      out_shape=jax.ShapeDtypeStruct((M, Dw), jnp.float32),
  )(x, W)


def _cmb_mm_body(p_ref, b_ref, w_ref, o_ref):
  g = jnp.maximum(p_ref[0] + p_ref[1] + b_ref[...], 0.0)
  o_ref[...] = jnp.dot(g, w_ref[...], preferred_element_type=jnp.float32)


def _cmb_mm(p, b, W):
  K = p.shape[2]
  Dw = W.shape[1]
  return pl.pallas_call(
      _cmb_mm_body,
      grid=(N // _BM,),
      in_specs=[pl.BlockSpec((NC, _BM, K), lambda i: (0, i, 0)),
                pl.BlockSpec((1, K), lambda i: (0, 0)),
                pl.BlockSpec((K, Dw), lambda i: (0, 0))],
      out_specs=pl.BlockSpec((_BM, Dw), lambda i: (i, 0)),
      out_shape=jax.ShapeDtypeStruct((N, Dw), jnp.float32),
  )(p, b, W)


def _fin_body(q_ref, b_ref, o_ref):
  s = q_ref[0] + q_ref[1]
  o_ref[...] = s[:, :D_OUT] + b_ref[...]


def _fin(q, b):
  return pl.pallas_call(
      _fin_body,
      grid=(N // _BM,),
      in_specs=[pl.BlockSpec((NC, _BM, D_OUT_PAD), lambda i: (0, i, 0)),
                pl.BlockSpec((1, D_OUT), lambda i: (0, 0))],
      out_specs=pl.BlockSpec((_BM, D_OUT), lambda i: (i, 0)),
      out_shape=jax.ShapeDtypeStruct((N, D_OUT), jnp.float32),
  )(q, b)


def kernel(x, edge_index, label_p, cm, W1, b1, W2, b2, W3, b3):
  z128 = jnp.zeros((N, D_HID), jnp.float32)
  z48 = jnp.zeros((N, D_OUT_PAD), jnp.float32)
  W3p = jnp.pad(W3, ((0, 0), (0, D_OUT_PAD - D_OUT)))

  hw1 = _mm(x, W1)
  p1 = _agg128(hw1, edge_index, z128)
  hw2 = _cmb_mm(p1, b1.reshape(1, -1), W2)
  p2 = _agg128(hw2, edge_index, z128)
  hw3 = _cmb_mm(p2, b2.reshape(1, -1), W3p)
  q = _agg48(hw3, edge_index, z48)
  return _fin(q, b3.reshape(1, -1))


# speedup vs baseline: 1.0874x; 1.0497x over previous
import jax
import jax.numpy as jnp
from jax import lax
from jax.experimental import pallas as pl
from jax.experimental.pallas import tpu as pltpu
from jax.experimental.pallas import tpu_sc as plsc

N = 10000
E = 320000
D_IN = 128
D_HID = 128
D_OUT = 40
D_OUT_PAD = 48

CH = 128
NCHUNK = E // CH
NC = 2
NS = 16
NW = NC * NS
CPW = NCHUNK // NW
TAIL = NCHUNK - CPW * NW

RT = 624
RT_LAST = N - RT * (NS - 1)


def _make_agg(D):
  mesh = plsc.VectorSubcoreMesh(core_axis_name="c", subcore_axis_name="s")

  def body(hw, ei, zeros, part,
           isv, idv, rows0, rows1, rows2, acc,
           ig0, ig1, ig2, ig3, ig4, ig5,
           gs0, gs1, gs2, ss0, ss1, ss2):
    cid = lax.axis_index("c")
    sid = lax.axis_index("s")
    wid = sid * NC + cid
    c0 = wid * CPW

    base = sid * RT
    rows = (rows0, rows1, rows2)
    gsem = (gs0, gs1, gs2)
    ssem = (ss0, ss1, ss2)
    isem = (ig0, ig1, ig2, ig3, ig4, ig5)

    def fire_i(c, q):
      pltpu.async_copy(ei.at[0, pl.ds(c * CH, CH)], isv.at[q], isem[q])
      pltpu.async_copy(ei.at[1, pl.ds(c * CH, CH)], idv.at[q], isem[q])

    def wait_i(q):
      pltpu.make_async_copy(ei.at[0, pl.ds(0, CH)], isv.at[q], isem[q]).wait()
      pltpu.make_async_copy(ei.at[1, pl.ds(0, CH)], idv.at[q], isem[q]).wait()

    def fire_g(q, b):
      pltpu.async_copy(hw.at[isv.at[q]], rows[b], gsem[b])

    def wait_g(b):
      pltpu.make_async_copy(hw.at[pl.ds(0, CH)], rows[b], gsem[b]).wait()

    def fire_s(q, b):
      pltpu.async_copy(rows[b], acc.at[idv.at[q]], ssem[b], add=True)

    def wait_s(b):
      pltpu.make_async_copy(rows[b], acc.at[pl.ds(0, CH)], ssem[b]).wait()

    for q in range(4):
      fire_i(c0 + q, q)
    wait_i(0)
    fire_g(0, 0)
    wait_i(1)
    fire_g(1, 1)

    @pl.when(sid < NS - 1)
    def _():
      pltpu.sync_copy(zeros.at[pl.ds(base, RT)], acc.at[pl.ds(base, RT)])

    @pl.when(sid == NS - 1)
    def _():
      pltpu.sync_copy(zeros.at[pl.ds(base, RT_LAST)], acc.at[pl.ds(base, RT_LAST)])

    plsc.subcore_barrier()

    def step(u, carry):
      for m in range(6):
        j = u * 6 + m
        b = m % 3

        wait_g(b)

        @pl.when(j > 0)
        def _():
          wait_s((m + 2) % 3)

        fire_s(m, b)

        @pl.when(j + 4 < CPW)
        def _():
          fire_i(c0 + j + 4, (m + 4) % 6)

        @pl.when(j + 2 < CPW)
        def _():
          wait_i((m + 2) % 6)
          fire_g((m + 2) % 6, (m + 2) % 3)

      return carry

    lax.fori_loop(0, CPW // 6, step, 0, unroll=False)
    wait_s((CPW - 1) % 3)

    @pl.when(wid < TAIL)
    def _():
      fire_i(NW * CPW + wid, 0)
      wait_i(0)
      fire_g(0, 0)
      wait_g(0)
      fire_s(0, 0)
      wait_s(0)

    plsc.subcore_barrier()

    @pl.when(sid < NS - 1)
    def _():
      pltpu.sync_copy(acc.at[pl.ds(base, RT)], part.at[cid, pl.ds(base, RT)])

    @pl.when(sid == NS - 1)
    def _():
      pltpu.sync_copy(acc.at[pl.ds(base, RT_LAST)],
                      part.at[cid, pl.ds(base, RT_LAST)])

  return pl.kernel(
      body,
      out_type=jax.ShapeDtypeStruct((NC, N, D), jnp.float32),
      mesh=mesh,
      compiler_params=pltpu.CompilerParams(use_tc_tiling_on_sc=False),
      scratch_types=(
          [pltpu.VMEM((6, CH), jnp.int32),
           pltpu.VMEM((6, CH), jnp.int32),
           pltpu.VMEM((CH, D), jnp.float32),
           pltpu.VMEM((CH, D), jnp.float32),
           pltpu.VMEM((CH, D), jnp.float32),
           pltpu.VMEM_SHARED((N, D), jnp.float32)]
          + [pltpu.SemaphoreType.DMA] * 12),
  )


_agg128 = _make_agg(D_HID)
_agg48 = _make_agg(D_OUT_PAD)

_BM = 1000


def _mm_body(x_ref, w_ref, o_ref):
  o_ref[...] = jnp.dot(x_ref[...], w_ref[...], preferred_element_type=jnp.float32)


def _mm(x, W):
  M, K = x.shape
  Dw = W.shape[1]
  return pl.pallas_call(
      _mm_body,
      grid=(M // _BM,),
      in_specs=[pl.BlockSpec((_BM, K), lambda i: (i, 0)),
                pl.BlockSpec((K, Dw), lambda i: (0, 0))],
      out_specs=pl.BlockSpec((_BM, Dw), lambda i: (i, 0)),
      out_shape=jax.ShapeDtypeStruct((M, Dw), jnp.float32),
  )(x, W)


def _cmb_mm_body(p_ref, b_ref, w_ref, o_ref):
  g = jnp.maximum(p_ref[0] + p_ref[1] + b_ref[...], 0.0)
  o_ref[...] = jnp.dot(g, w_ref[...], preferred_element_type=jnp.float32)


def _cmb_mm(p, b, W):
  K = p.shape[2]
  Dw = W.shape[1]
  return pl.pallas_call(
      _cmb_mm_body,
      grid=(N // _BM,),
      in_specs=[pl.BlockSpec((NC, _BM, K), lambda i: (0, i, 0)),
                pl.BlockSpec((1, K), lambda i: (0, 0)),
                pl.BlockSpec((K, Dw), lambda i: (0, 0))],
      out_specs=pl.BlockSpec((_BM, Dw), lambda i: (i, 0)),
      out_shape=jax.ShapeDtypeStruct((N, Dw), jnp.float32),
  )(p, b, W)


def _fin_body(q_ref, b_ref, o_ref):
  s = q_ref[0] + q_ref[1]
  o_ref[...] = s[:, :D_OUT] + b_ref[...]


def _fin(q, b):
  return pl.pallas_call(
      _fin_body,
      grid=(N // _BM,),
      in_specs=[pl.BlockSpec((NC, _BM, D_OUT_PAD), lambda i: (0, i, 0)),
                pl.BlockSpec((1, D_OUT), lambda i: (0, 0))],
      out_specs=pl.BlockSpec((_BM, D_OUT), lambda i: (i, 0)),
      out_shape=jax.ShapeDtypeStruct((N, D_OUT), jnp.float32),
  )(q, b)


def kernel(x, edge_index, label_p, cm, W1, b1, W2, b2, W3, b3):
  z128 = jnp.zeros((N, D_HID), jnp.float32)
  z48 = jnp.zeros((N, D_OUT_PAD), jnp.float32)
  W3p = jnp.pad(W3, ((0, 0), (0, D_OUT_PAD - D_OUT)))

  hw1 = _mm(x, W1)
  p1 = _agg128(hw1, edge_index, z128)
  hw2 = _cmb_mm(p1, b1.reshape(1, -1), W2)
  p2 = _agg128(hw2, edge_index, z128)
  hw3 = _cmb_mm(p2, b2.reshape(1, -1), W3p)
  q = _agg48(hw3, edge_index, z48)
  return _fin(q, b3.reshape(1, -1))
